# Initial kernel scaffold; baseline (speedup 1.0000x reference)
import jax
import jax.numpy as jnp
from jax.experimental import pallas as pl


def kernel(x, edge_index, W1l, b1, W1r, W2l, b2, W2r, Wlin, blin):
    raise NotImplementedError("write your pallas kernel here")


# speedup vs baseline: 3.7535x; 3.7535x over previous
import jax
import jax.numpy as jnp
from jax import lax
from jax.experimental import pallas as pl
from jax.experimental.pallas import tpu as pltpu
from jax.experimental.pallas import tpu_sc as plsc

N = 10000
E = 320000
F_IN = 128
H = 256
C_OUT = 64

NC = 2
NS = 16
NW = NC * NS

CHUNK = 80
N_ROW_CHUNKS = N // CHUNK
RC_PER_TILE = -(-N_ROW_CHUNKS // NS)

_mesh = plsc.VectorSubcoreMesh(core_axis_name="c", subcore_axis_name="s")


def _zero_buf(buf, rows, width):
    lanes = width // 16

    def st(j, carry):
        buf[j // lanes, pl.ds(pl.multiple_of((j % lanes) * 16, 16), 16)] = (
            jnp.zeros((16,), jnp.float32))
        return carry
    lax.fori_loop(0, rows * lanes, st, 0)


def _row_chunk0(s, j):
    q = jnp.minimum(s * RC_PER_TILE + j, N_ROW_CHUNKS - 1)
    return pl.multiple_of(q * CHUNK, 8)


def _fill_shared_zero(s, buf, sh_ref):
    def cp(j, carry):
        r = _row_chunk0(s, j)
        pltpu.sync_copy(buf, sh_ref.at[pl.ds(r, CHUNK)])
        return carry
    lax.fori_loop(0, RC_PER_TILE, cp, 0)


def _drain_shared(c, s, buf, sh_ref, out_ref):
    def cp(j, carry):
        r = _row_chunk0(s, j)
        pltpu.sync_copy(sh_ref.at[pl.ds(r, CHUNK)], buf)
        pltpu.sync_copy(buf, out_ref.at[c, pl.ds(r, CHUNK), :])
        return carry
    lax.fori_loop(0, RC_PER_TILE, cp, 0)


def _sc_agg_x_body(src_ref, dst_ref, x_ref,
                   agg_out, cnt_out,
                   src_i, dst_i, rows_v, acc_sh, sem):
    c = lax.axis_index("c")
    s = lax.axis_index("s")
    w = c * NS + s

    _zero_buf(rows_v, CHUNK, F_IN)
    _fill_shared_zero(s, rows_v, acc_sh)
    plsc.subcore_barrier()

    base = w * (E // NW)
    n_chunks = (E // NW) // CHUNK

    def body(i, carry):
        off = pl.multiple_of(base + i * CHUNK, 8)
        pltpu.sync_copy(src_ref.at[pl.ds(off, CHUNK)], src_i)
        pltpu.sync_copy(dst_ref.at[pl.ds(off, CHUNK)], dst_i)
        pltpu.async_copy(x_ref.at[src_i], rows_v, sem).wait()
        pltpu.sync_copy(rows_v, acc_sh.at[dst_i], add=True)
        return carry
    lax.fori_loop(0, n_chunks, body, 0)

    plsc.subcore_barrier()
    _drain_shared(c, s, rows_v, acc_sh, agg_out)
    plsc.subcore_barrier()

    _zero_buf(rows_v, CHUNK, F_IN)
    _fill_shared_zero(s, rows_v, acc_sh)
    plsc.subcore_barrier()

    def init_ones(j, carry):
        rows_v[j // (F_IN // 16),
               pl.ds(pl.multiple_of((j % (F_IN // 16)) * 16, 16), 16)] = (
            jnp.ones((16,), jnp.float32))
        return carry
    lax.fori_loop(0, CHUNK * (F_IN // 16), init_ones, 0)

    def cbody(i, carry):
        off = pl.multiple_of(base + i * CHUNK, 8)
        pltpu.sync_copy(dst_ref.at[pl.ds(off, CHUNK)], dst_i)
        pltpu.sync_copy(rows_v, acc_sh.at[dst_i], add=True)
        return carry
    lax.fori_loop(0, n_chunks, cbody, 0)

    plsc.subcore_barrier()
    _drain_shared(c, s, rows_v, acc_sh, cnt_out)


_sc_agg_x = pl.kernel(
    _sc_agg_x_body,
    out_type=[jax.ShapeDtypeStruct((NC, N, F_IN), jnp.float32),
              jax.ShapeDtypeStruct((NC, N, F_IN), jnp.float32)],
    mesh=_mesh,
    scratch_types=[
        pltpu.VMEM((CHUNK,), jnp.int32),
        pltpu.VMEM((CHUNK,), jnp.int32),
        pltpu.VMEM((CHUNK, F_IN), jnp.float32),
        pltpu.VMEM_SHARED((N, F_IN), jnp.float32),
        pltpu.SemaphoreType.DMA,
    ],
)


def _sc_agg_h_body(src_ref, dst_ref, hcat_ref,
                   agg_out,
                   src_i, dst_i, rows_v, acc_sh, sem):
    c = lax.axis_index("c")
    s = lax.axis_index("s")

    _zero_buf(rows_v, CHUNK, H // 2)
    _fill_shared_zero(s, rows_v, acc_sh)
    plsc.subcore_barrier()

    base = s * (E // NS)
    n_chunks = (E // NS) // CHUNK
    cN = c * N

    def body(i, carry):
        off = pl.multiple_of(base + i * CHUNK, 8)
        pltpu.sync_copy(src_ref.at[pl.ds(off, CHUNK)], src_i)
        pltpu.sync_copy(dst_ref.at[pl.ds(off, CHUNK)], dst_i)

        def addoff(k, carry2):
            o = pl.ds(pl.multiple_of(k * 16, 16), 16)
            src_i[o] = src_i[o] + cN
            return carry2
        lax.fori_loop(0, CHUNK // 16, addoff, 0)

        pltpu.async_copy(hcat_ref.at[src_i], rows_v, sem).wait()
        pltpu.sync_copy(rows_v, acc_sh.at[dst_i], add=True)
        return carry
    lax.fori_loop(0, n_chunks, body, 0)

    plsc.subcore_barrier()
    _drain_shared(c, s, rows_v, acc_sh, agg_out)


_sc_agg_h = pl.kernel(
    _sc_agg_h_body,
    out_type=jax.ShapeDtypeStruct((NC, N, H // 2), jnp.float32),
    mesh=_mesh,
    scratch_types=[
        pltpu.VMEM((CHUNK,), jnp.int32),
        pltpu.VMEM((CHUNK,), jnp.int32),
        pltpu.VMEM((CHUNK, H // 2), jnp.float32),
        pltpu.VMEM_SHARED((N, H // 2), jnp.float32),
        pltpu.SemaphoreType.DMA,
    ],
)


BR = 1000


def _dotT(a, w):
    return lax.dot_general(a, w, (((1,), (1,)), ((), ())),
                           preferred_element_type=jnp.float32)


def _tc_layer1_body(agg_ref, cnt_ref, x_ref, w1l_ref, b1_ref, w1r_ref,
                    h_ref):
    cnt = cnt_ref[0, :, 0:1] + cnt_ref[1, :, 0:1]
    denom = jnp.maximum(cnt, 1.0)
    agg = (agg_ref[0] + agg_ref[1]) / denom
    t = _dotT(agg, w1l_ref[...]) + b1_ref[...] + _dotT(x_ref[...], w1r_ref[...])
    h = jnp.maximum(t, 0.0)
    h_ref[0] = h[:, :H // 2]
    h_ref[1] = h[:, H // 2:]


def _tc_layer2_body(agg2_ref, cnt_ref, h_ref,
                    w2l_ref, b2_ref, w2r_ref, wlin_ref, blin_ref,
                    out_ref):
    cnt = cnt_ref[0, :, 0:1] + cnt_ref[1, :, 0:1]
    denom = jnp.maximum(cnt, 1.0)
    aggcat = jnp.concatenate([agg2_ref[0], agg2_ref[1]], axis=1) / denom
    hcat = jnp.concatenate([h_ref[0], h_ref[1]], axis=1)
    t = _dotT(aggcat, w2l_ref[...]) + b2_ref[...] + _dotT(hcat, w2r_ref[...])
    hh = hcat + t
    logits = _dotT(hh, wlin_ref[...]) + blin_ref[...]
    m = jnp.max(logits, axis=1, keepdims=True)
    lse = jnp.log(jnp.sum(jnp.exp(logits - m), axis=1, keepdims=True)) + m
    out_ref[...] = logits - lse


def _whole(shape):
    return pl.BlockSpec(shape, lambda i: tuple(0 for _ in shape))


def kernel(x, edge_index, W1l, b1, W1r, W2l, b2, W2r, Wlin, blin):
    src = edge_index[0].astype(jnp.int32)
    dst = edge_index[1].astype(jnp.int32)

    agg1, cnt1 = _sc_agg_x(src, dst, x)

    b1r = b1.reshape(1, H)
    b2r = b2.reshape(1, H)
    blinr = blin.reshape(1, C_OUT)

    grid = (N // BR,)
    h2 = pl.pallas_call(
        _tc_layer1_body,
        grid=grid,
        in_specs=[
            pl.BlockSpec((NC, BR, F_IN), lambda i: (0, i, 0)),
            pl.BlockSpec((NC, BR, F_IN), lambda i: (0, i, 0)),
            pl.BlockSpec((BR, F_IN), lambda i: (i, 0)),
            _whole((H, F_IN)), _whole((1, H)), _whole((H, F_IN)),
        ],
        out_specs=pl.BlockSpec((NC, BR, H // 2), lambda i: (0, i, 0)),
        out_shape=jax.ShapeDtypeStruct((NC, N, H // 2), jnp.float32),
    )(agg1, cnt1, x, W1l, b1r, W1r)

    hcat = h2.reshape(NC * N, H // 2)
    agg2 = _sc_agg_h(src, dst, hcat)

    out = pl.pallas_call(
        _tc_layer2_body,
        grid=grid,
        in_specs=[
            pl.BlockSpec((NC, BR, H // 2), lambda i: (0, i, 0)),
            pl.BlockSpec((NC, BR, F_IN), lambda i: (0, i, 0)),
            pl.BlockSpec((NC, BR, H // 2), lambda i: (0, i, 0)),
            _whole((H, H)), _whole((1, H)), _whole((H, H)),
            _whole((C_OUT, H)), _whole((1, C_OUT)),
        ],
        out_specs=pl.BlockSpec((BR, C_OUT), lambda i: (i, 0)),
        out_shape=jax.ShapeDtypeStruct((N, C_OUT), jnp.float32),
    )(agg2, cnt1, h2, W2l, b2r, W2r, Wlin, blinr)

    return (out, edge_index)


# speedup vs baseline: 7.0775x; 1.8855x over previous
import jax
import jax.numpy as jnp
from jax import lax
from jax.experimental import pallas as pl
from jax.experimental.pallas import tpu as pltpu
from jax.experimental.pallas import tpu_sc as plsc

N = 10000
E = 320000
F_IN = 128
H = 256
C_OUT = 64

NC = 2
NS = 16
NW = NC * NS

CHUNK = 80
N_ROW_CHUNKS = N // CHUNK
RC_PER_TILE = -(-N_ROW_CHUNKS // NS)
DEPTH = 3

_mesh = plsc.VectorSubcoreMesh(core_axis_name="c", subcore_axis_name="s")


def _zero_buf(buf, rows, width):
    lanes = width // 16

    def st(j, carry):
        buf[j // lanes, pl.ds(pl.multiple_of((j % lanes) * 16, 16), 16)] = (
            jnp.zeros((16,), jnp.float32))
        return carry
    lax.fori_loop(0, rows * lanes, st, 0)


def _ones_buf(buf, rows, width):
    lanes = width // 16

    def st(j, carry):
        buf[j // lanes, pl.ds(pl.multiple_of((j % lanes) * 16, 16), 16)] = (
            jnp.ones((16,), jnp.float32))
        return carry
    lax.fori_loop(0, rows * lanes, st, 0)


def _row_chunk0(s, j):
    q = jnp.minimum(s * RC_PER_TILE + j, N_ROW_CHUNKS - 1)
    return pl.multiple_of(q * CHUNK, 8)


def _fill_shared_zero(s, buf, sh_ref, sem):
    for j in range(RC_PER_TILE):
        r = _row_chunk0(s, j)
        pltpu.async_copy(buf, sh_ref.at[pl.ds(r, CHUNK)], sem)
    for j in range(RC_PER_TILE):
        r = _row_chunk0(s, j)
        pltpu.make_async_copy(buf, sh_ref.at[pl.ds(r, CHUNK)], sem).wait()


def _drain_shared(c, s, bufs, sems, sh_ref, out_ref):
    nb = 2

    def pull(q):
        b = q % nb
        r = _row_chunk0(s, q)
        return pltpu.async_copy(sh_ref.at[pl.ds(r, CHUNK)], bufs[b], sems[b])

    def push(q):
        b = q % nb
        r = _row_chunk0(s, q)
        return pltpu.async_copy(bufs[b], out_ref.at[c, pl.ds(r, CHUNK), :],
                                sems[b])

    pend = {}
    pend[0] = pull(0)
    pend[1] = pull(1)
    for q in range(RC_PER_TILE):
        pend.pop(q).wait()
        dsc = push(q)
        if q + nb < RC_PER_TILE:
            dsc.wait()
            pend[q + nb] = pull(q + nb)
        else:
            pend[q] = dsc
    for q in sorted(pend):
        pend[q].wait()


def _edge_pipeline(n, base, dst_ref, dst_bufs, sem_d, acc_sh, scat_src, sem_sc,
                   src_ref=None, src_bufs=None, sem_s=None,
                   table_ref=None, rows_bufs=None, sem_g=None, idx_add=None):
    gather = src_ref is not None

    def off_of(j):
        return pl.multiple_of(base + j * CHUNK, 8)

    def emitB(j, b):
        off = off_of(j)
        if gather:
            pltpu.async_copy(src_ref.at[pl.ds(off, CHUNK)], src_bufs[b], sem_s[b])
        pltpu.async_copy(dst_ref.at[pl.ds(off, CHUNK)], dst_bufs[b], sem_d[b])

    def emitA(b):
        pltpu.make_async_copy(scat_src(b), acc_sh.at[dst_bufs[b]], sem_sc[b]).wait()

    def emitC(j, b):
        off = off_of(j)
        if gather:
            pltpu.make_async_copy(src_ref.at[pl.ds(off, CHUNK)], src_bufs[b],
                                  sem_s[b]).wait()
            if idx_add is not None:
                def addo(k, carry):
                    o = pl.ds(pl.multiple_of(k * 16, 16), 16)
                    src_bufs[b][o] = src_bufs[b][o] + idx_add
                    return carry
                lax.fori_loop(0, CHUNK // 16, addo, 0)
            pltpu.async_copy(table_ref.at[src_bufs[b]], rows_bufs[b], sem_g).wait()
        pltpu.make_async_copy(dst_ref.at[pl.ds(off, CHUNK)], dst_bufs[b],
                              sem_d[b]).wait()
        pltpu.async_copy(scat_src(b), acc_sh.at[dst_bufs[b]], sem_sc[b], add=True)

    M = (n - 2) // DEPTH
    emitB(0, 0)
    emitB(1, 1)
    emitC(0, 0); emitB(2, 2)
    emitC(1, 1); emitA(0); emitB(3, 0)
    emitC(2, 2); emitA(1); emitB(4, 1)

    def body(t, carry):
        for k in range(DEPTH):
            j = t * DEPTH + k
            emitC(j, k)
            emitA((k + 2) % DEPTH)
            emitB(j + 2, (k + 2) % DEPTH)
        return carry
    lax.fori_loop(1, M, body, 0)

    for j in range(DEPTH * M, n):
        if j >= DEPTH * M + 2:
            emitA(j % DEPTH)
            emitB(j, j % DEPTH)
        emitC(j, j % DEPTH)
    for j in range(n - DEPTH, n):
        emitA(j % DEPTH)


def _sc_agg_x_body(src_ref, dst_ref, x_ref,
                   agg_out, cnt_out,
                   si0, si1, si2, di0, di1, di2, rv0, rv1, rv2, acc_sh,
                   sems0, sems1, sems2, semd0, semd1, semd2,
                   semc0, semc1, semc2, semg):
    c = lax.axis_index("c")
    s = lax.axis_index("s")
    w = c * NS + s
    src_bufs = (si0, si1, si2)
    dst_bufs = (di0, di1, di2)
    rows_bufs = (rv0, rv1, rv2)
    sem_s = (sems0, sems1, sems2)
    sem_d = (semd0, semd1, semd2)
    sem_sc = (semc0, semc1, semc2)

    _zero_buf(rv0, CHUNK, F_IN)
    _fill_shared_zero(s, rv0, acc_sh, semg)
    plsc.subcore_barrier()

    base = w * (E // NW)
    n_chunks = (E // NW) // CHUNK

    _edge_pipeline(n_chunks, base, dst_ref, dst_bufs, sem_d, acc_sh,
                   lambda b: rows_bufs[b], sem_sc,
                   src_ref=src_ref, src_bufs=src_bufs, sem_s=sem_s,
                   table_ref=x_ref, rows_bufs=rows_bufs, sem_g=semg)

    plsc.subcore_barrier()
    _drain_shared(c, s, (rv0, rv1), (semg, semc0), acc_sh, agg_out)
    plsc.subcore_barrier()

    _zero_buf(rv0, CHUNK, F_IN)
    _fill_shared_zero(s, rv0, acc_sh, semg)
    plsc.subcore_barrier()

    _ones_buf(rv0, CHUNK, F_IN)
    _edge_pipeline(n_chunks, base, dst_ref, dst_bufs, sem_d, acc_sh,
                   lambda b: rv0, sem_sc)

    plsc.subcore_barrier()
    _drain_shared(c, s, (rv1, rv2), (semg, semc0), acc_sh, cnt_out)


_sc_agg_x = pl.kernel(
    _sc_agg_x_body,
    out_type=[jax.ShapeDtypeStruct((NC, N, F_IN), jnp.float32),
              jax.ShapeDtypeStruct((NC, N, F_IN), jnp.float32)],
    mesh=_mesh,
    scratch_types=(
        [pltpu.VMEM((CHUNK,), jnp.int32)] * 3
        + [pltpu.VMEM((CHUNK,), jnp.int32)] * 3
        + [pltpu.VMEM((CHUNK, F_IN), jnp.float32)] * 3
        + [pltpu.VMEM_SHARED((N, F_IN), jnp.float32)]
        + [pltpu.SemaphoreType.DMA] * 10
    ),
)


def _sc_agg_h_body(src_ref, dst_ref, hcat_ref,
                   agg_out,
                   si0, si1, si2, di0, di1, di2, rv0, rv1, rv2, acc_sh,
                   sems0, sems1, sems2, semd0, semd1, semd2,
                   semc0, semc1, semc2, semg):
    c = lax.axis_index("c")
    s = lax.axis_index("s")
    src_bufs = (si0, si1, si2)
    dst_bufs = (di0, di1, di2)
    rows_bufs = (rv0, rv1, rv2)
    sem_s = (sems0, sems1, sems2)
    sem_d = (semd0, semd1, semd2)
    sem_sc = (semc0, semc1, semc2)

    _zero_buf(rv0, CHUNK, H // 2)
    _fill_shared_zero(s, rv0, acc_sh, semg)
    plsc.subcore_barrier()

    base = s * (E // NS)
    n_chunks = (E // NS) // CHUNK

    _edge_pipeline(n_chunks, base, dst_ref, dst_bufs, sem_d, acc_sh,
                   lambda b: rows_bufs[b], sem_sc,
                   src_ref=src_ref, src_bufs=src_bufs, sem_s=sem_s,
                   table_ref=hcat_ref, rows_bufs=rows_bufs, sem_g=semg,
                   idx_add=c * N)

    plsc.subcore_barrier()
    _drain_shared(c, s, (rv0, rv1), (semg, semc0), acc_sh, agg_out)


_sc_agg_h = pl.kernel(
    _sc_agg_h_body,
    out_type=jax.ShapeDtypeStruct((NC, N, H // 2), jnp.float32),
    mesh=_mesh,
    scratch_types=(
        [pltpu.VMEM((CHUNK,), jnp.int32)] * 3
        + [pltpu.VMEM((CHUNK,), jnp.int32)] * 3
        + [pltpu.VMEM((CHUNK, H // 2), jnp.float32)] * 3
        + [pltpu.VMEM_SHARED((N, H // 2), jnp.float32)]
        + [pltpu.SemaphoreType.DMA] * 10
    ),
)


BR = 1000


def _dotT(a, w):
    return lax.dot_general(a, w, (((1,), (1,)), ((), ())),
                           preferred_element_type=jnp.float32)


def _tc_layer1_body(agg_ref, cnt_ref, x_ref, w1l_ref, b1_ref, w1r_ref,
                    h_ref):
    cnt = cnt_ref[0, :, 0:1] + cnt_ref[1, :, 0:1]
    denom = jnp.maximum(cnt, 1.0)
    agg = (agg_ref[0] + agg_ref[1]) / denom
    t = _dotT(agg, w1l_ref[...]) + b1_ref[...] + _dotT(x_ref[...], w1r_ref[...])
    h = jnp.maximum(t, 0.0)
    h_ref[0] = h[:, :H // 2]
    h_ref[1] = h[:, H // 2:]


def _tc_layer2_body(agg2_ref, cnt_ref, h_ref,
                    w2l_ref, b2_ref, w2r_ref, wlin_ref, blin_ref,
                    out_ref):
    cnt = cnt_ref[0, :, 0:1] + cnt_ref[1, :, 0:1]
    denom = jnp.maximum(cnt, 1.0)
    aggcat = jnp.concatenate([agg2_ref[0], agg2_ref[1]], axis=1) / denom
    hcat = jnp.concatenate([h_ref[0], h_ref[1]], axis=1)
    t = _dotT(aggcat, w2l_ref[...]) + b2_ref[...] + _dotT(hcat, w2r_ref[...])
    hh = hcat + t
    logits = _dotT(hh, wlin_ref[...]) + blin_ref[...]
    m = jnp.max(logits, axis=1, keepdims=True)
    lse = jnp.log(jnp.sum(jnp.exp(logits - m), axis=1, keepdims=True)) + m
    out_ref[...] = logits - lse


def _whole(shape):
    return pl.BlockSpec(shape, lambda i: tuple(0 for _ in shape))


def kernel(x, edge_index, W1l, b1, W1r, W2l, b2, W2r, Wlin, blin):
    src = edge_index[0].astype(jnp.int32)
    dst = edge_index[1].astype(jnp.int32)

    agg1, cnt1 = _sc_agg_x(src, dst, x)

    b1r = b1.reshape(1, H)
    b2r = b2.reshape(1, H)
    blinr = blin.reshape(1, C_OUT)

    grid = (N // BR,)
    h2 = pl.pallas_call(
        _tc_layer1_body,
        grid=grid,
        in_specs=[
            pl.BlockSpec((NC, BR, F_IN), lambda i: (0, i, 0)),
            pl.BlockSpec((NC, BR, F_IN), lambda i: (0, i, 0)),
            pl.BlockSpec((BR, F_IN), lambda i: (i, 0)),
            _whole((H, F_IN)), _whole((1, H)), _whole((H, F_IN)),
        ],
        out_specs=pl.BlockSpec((NC, BR, H // 2), lambda i: (0, i, 0)),
        out_shape=jax.ShapeDtypeStruct((NC, N, H // 2), jnp.float32),
    )(agg1, cnt1, x, W1l, b1r, W1r)

    hcat = h2.reshape(NC * N, H // 2)
    agg2 = _sc_agg_h(src, dst, hcat)

    out = pl.pallas_call(
        _tc_layer2_body,
        grid=grid,
        in_specs=[
            pl.BlockSpec((NC, BR, H // 2), lambda i: (0, i, 0)),
            pl.BlockSpec((NC, BR, F_IN), lambda i: (0, i, 0)),
            pl.BlockSpec((NC, BR, H // 2), lambda i: (0, i, 0)),
            _whole((H, H)), _whole((1, H)), _whole((H, H)),
            _whole((C_OUT, H)), _whole((1, C_OUT)),
        ],
        out_specs=pl.BlockSpec((BR, C_OUT), lambda i: (i, 0)),
        out_shape=jax.ShapeDtypeStruct((N, C_OUT), jnp.float32),
    )(agg2, cnt1, h2, W2l, b2r, W2r, Wlin, blinr)

    return (out, edge_index)


# speedup vs baseline: 9.2841x; 1.3118x over previous
import jax
import jax.numpy as jnp
from jax import lax
from jax.experimental import pallas as pl
from jax.experimental.pallas import tpu as pltpu
from jax.experimental.pallas import tpu_sc as plsc

N = 10000
E = 320000
F_IN = 128
H = 256
C_OUT = 64

NC = 2
NS = 16
NW = NC * NS

CHUNK = 80
N_ROW_CHUNKS = N // CHUNK
RC_PER_TILE = -(-N_ROW_CHUNKS // NS)
DEPTH = 3

_mesh = plsc.VectorSubcoreMesh(core_axis_name="c", subcore_axis_name="s")


def _zero_buf(buf, rows, width):
    lanes = width // 16

    def st(j, carry):
        buf[j // lanes, pl.ds(pl.multiple_of((j % lanes) * 16, 16), 16)] = (
            jnp.zeros((16,), jnp.float32))
        return carry
    lax.fori_loop(0, rows * lanes, st, 0)


def _ones_buf(buf, rows, width):
    lanes = width // 16

    def st(j, carry):
        buf[j // lanes, pl.ds(pl.multiple_of((j % lanes) * 16, 16), 16)] = (
            jnp.ones((16,), jnp.float32))
        return carry
    lax.fori_loop(0, rows * lanes, st, 0)


def _row_chunk0(s, j):
    q = jnp.minimum(s * RC_PER_TILE + j, N_ROW_CHUNKS - 1)
    return pl.multiple_of(q * CHUNK, 8)


def _fill_shared_zero(s, buf, sh_ref, sem):
    for j in range(RC_PER_TILE):
        r = _row_chunk0(s, j)
        pltpu.async_copy(buf, sh_ref.at[pl.ds(r, CHUNK)], sem)
    for j in range(RC_PER_TILE):
        r = _row_chunk0(s, j)
        pltpu.make_async_copy(buf, sh_ref.at[pl.ds(r, CHUNK)], sem).wait()


def _drain_shared(c, s, bufs, sems, sh_ref, out_ref):
    nb = 2

    def pull(q):
        b = q % nb
        r = _row_chunk0(s, q)
        return pltpu.async_copy(sh_ref.at[pl.ds(r, CHUNK)], bufs[b], sems[b])

    def push(q):
        b = q % nb
        r = _row_chunk0(s, q)
        return pltpu.async_copy(bufs[b], out_ref.at[c, pl.ds(r, CHUNK), :],
                                sems[b])

    pend = {}
    pend[0] = pull(0)
    pend[1] = pull(1)
    for q in range(RC_PER_TILE):
        pend.pop(q).wait()
        dsc = push(q)
        if q + nb < RC_PER_TILE:
            dsc.wait()
            pend[q + nb] = pull(q + nb)
        else:
            pend[q] = dsc
    for q in sorted(pend):
        pend[q].wait()


def _edge_pipeline(n, base, dst_ref, dst_bufs, sem_d, acc_sh, scat_src, sem_sc,
                   src_ref=None, src_bufs=None, sem_s=None,
                   table_ref=None, rows_bufs=None, sem_g=None, idx_add=None):
    gather = src_ref is not None

    def off_of(j):
        return pl.multiple_of(base + j * CHUNK, 8)

    def emitB(j, b):
        off = off_of(j)
        if gather:
            pltpu.async_copy(src_ref.at[pl.ds(off, CHUNK)], src_bufs[b], sem_s[b])
        pltpu.async_copy(dst_ref.at[pl.ds(off, CHUNK)], dst_bufs[b], sem_d[b])

    def emitA(b):
        pltpu.make_async_copy(scat_src(b), acc_sh.at[dst_bufs[b]], sem_sc[b]).wait()

    def emitC(j, b):
        off = off_of(j)
        if gather:
            pltpu.make_async_copy(src_ref.at[pl.ds(off, CHUNK)], src_bufs[b],
                                  sem_s[b]).wait()
            if idx_add is not None:
                def addo(k, carry):
                    o = pl.ds(pl.multiple_of(k * 16, 16), 16)
                    src_bufs[b][o] = src_bufs[b][o] + idx_add
                    return carry
                lax.fori_loop(0, CHUNK // 16, addo, 0)
            pltpu.async_copy(table_ref.at[src_bufs[b]], rows_bufs[b], sem_g).wait()
        pltpu.make_async_copy(dst_ref.at[pl.ds(off, CHUNK)], dst_bufs[b],
                              sem_d[b]).wait()
        pltpu.async_copy(scat_src(b), acc_sh.at[dst_bufs[b]], sem_sc[b], add=True)

    M = (n - 2) // DEPTH
    emitB(0, 0)
    emitB(1, 1)
    emitC(0, 0); emitB(2, 2)
    emitC(1, 1); emitA(0); emitB(3, 0)
    emitC(2, 2); emitA(1); emitB(4, 1)

    def body(t, carry):
        for k in range(DEPTH):
            j = t * DEPTH + k
            emitC(j, k)
            emitA((k + 2) % DEPTH)
            emitB(j + 2, (k + 2) % DEPTH)
        return carry
    lax.fori_loop(1, M, body, 0)

    for j in range(DEPTH * M, n):
        if j >= DEPTH * M + 2:
            emitA(j % DEPTH)
            emitB(j, j % DEPTH)
        emitC(j, j % DEPTH)
    for j in range(n - DEPTH, n):
        emitA(j % DEPTH)


def _sc_agg_x_body(src_ref, dst_ref, x_ref,
                   agg_out, cnt_out,
                   si0, si1, si2, di0, di1, di2, rv0, rv1, rv2, acc_sh,
                   sems0, sems1, sems2, semd0, semd1, semd2,
                   semc0, semc1, semc2, semg):
    c = lax.axis_index("c")
    s = lax.axis_index("s")
    w = c * NS + s
    src_bufs = (si0, si1, si2)
    dst_bufs = (di0, di1, di2)
    rows_bufs = (rv0, rv1, rv2)
    sem_s = (sems0, sems1, sems2)
    sem_d = (semd0, semd1, semd2)
    sem_sc = (semc0, semc1, semc2)

    _zero_buf(rv0, CHUNK, F_IN)
    _fill_shared_zero(s, rv0, acc_sh, semg)
    plsc.subcore_barrier()

    base = w * (E // NW)
    n_chunks = (E // NW) // CHUNK

    _edge_pipeline(n_chunks, base, dst_ref, dst_bufs, sem_d, acc_sh,
                   lambda b: rows_bufs[b], sem_sc,
                   src_ref=src_ref, src_bufs=src_bufs, sem_s=sem_s,
                   table_ref=x_ref, rows_bufs=rows_bufs, sem_g=semg)

    plsc.subcore_barrier()
    _drain_shared(c, s, (rv0, rv1), (semg, semc0), acc_sh, agg_out)
    plsc.subcore_barrier()

    _zero_buf(rv0, CHUNK, F_IN)
    _fill_shared_zero(s, rv0, acc_sh, semg)
    plsc.subcore_barrier()

    _ones_buf(rv0, CHUNK, F_IN)
    _edge_pipeline(n_chunks, base, dst_ref, dst_bufs, sem_d, acc_sh,
                   lambda b: rv0, sem_sc)

    plsc.subcore_barrier()
    _drain_shared(c, s, (rv1, rv2), (semg, semc0), acc_sh, cnt_out)


_sc_agg_x = pl.kernel(
    _sc_agg_x_body,
    out_type=[jax.ShapeDtypeStruct((NC, N, F_IN), jnp.float32),
              jax.ShapeDtypeStruct((NC, N, F_IN), jnp.float32)],
    mesh=_mesh,
    scratch_types=(
        [pltpu.VMEM((CHUNK,), jnp.int32)] * 3
        + [pltpu.VMEM((CHUNK,), jnp.int32)] * 3
        + [pltpu.VMEM((CHUNK, F_IN), jnp.float32)] * 3
        + [pltpu.VMEM_SHARED((N, F_IN), jnp.float32)]
        + [pltpu.SemaphoreType.DMA] * 10
    ),
)


def _sc_agg_p_body(src_ref, dst_ref, hp_ref,
                   agg_out,
                   si0, si1, si2, di0, di1, di2, rv0, rv1, rv2, acc_sh,
                   sems0, sems1, sems2, semd0, semd1, semd2,
                   semc0, semc1, semc2, semg):
    c = lax.axis_index("c")
    s = lax.axis_index("s")
    w = c * NS + s
    src_bufs = (si0, si1, si2)
    dst_bufs = (di0, di1, di2)
    rows_bufs = (rv0, rv1, rv2)
    sem_s = (sems0, sems1, sems2)
    sem_d = (semd0, semd1, semd2)
    sem_sc = (semc0, semc1, semc2)

    _zero_buf(rv0, CHUNK, F_IN)
    _fill_shared_zero(s, rv0, acc_sh, semg)
    plsc.subcore_barrier()

    base = w * (E // NW)
    n_chunks = (E // NW) // CHUNK

    _edge_pipeline(n_chunks, base, dst_ref, dst_bufs, sem_d, acc_sh,
                   lambda b: rows_bufs[b], sem_sc,
                   src_ref=src_ref, src_bufs=src_bufs, sem_s=sem_s,
                   table_ref=hp_ref, rows_bufs=rows_bufs, sem_g=semg)

    plsc.subcore_barrier()
    _drain_shared(c, s, (rv0, rv1), (semg, semc0), acc_sh, agg_out)


_sc_agg_p = pl.kernel(
    _sc_agg_p_body,
    out_type=jax.ShapeDtypeStruct((NC, N, F_IN), jnp.float32),
    mesh=_mesh,
    scratch_types=(
        [pltpu.VMEM((CHUNK,), jnp.int32)] * 3
        + [pltpu.VMEM((CHUNK,), jnp.int32)] * 3
        + [pltpu.VMEM((CHUNK, F_IN), jnp.float32)] * 3
        + [pltpu.VMEM_SHARED((N, F_IN), jnp.float32)]
        + [pltpu.SemaphoreType.DMA] * 10
    ),
)


BR = 1000


def _dotT(a, w):
    return lax.dot_general(a, w, (((1,), (1,)), ((), ())),
                           preferred_element_type=jnp.float32)


def _tc_layer1_body(agg_ref, cnt_ref, x_ref, w1l_ref, b1_ref, w1r_ref,
                    w2l_ref, wlin_ref,
                    h_ref, hp_ref):
    cnt = cnt_ref[0, :, 0:1] + cnt_ref[1, :, 0:1]
    denom = jnp.maximum(cnt, 1.0)
    agg = (agg_ref[0] + agg_ref[1]) / denom
    t = _dotT(agg, w1l_ref[...]) + b1_ref[...] + _dotT(x_ref[...], w1r_ref[...])
    h = jnp.maximum(t, 0.0)
    h_ref[0] = h[:, :H // 2]
    h_ref[1] = h[:, H // 2:]
    wcomb = lax.dot_general(wlin_ref[...], w2l_ref[...],
                            (((1,), (0,)), ((), ())),
                            preferred_element_type=jnp.float32)
    hp = _dotT(h, wcomb)
    hp_ref[...] = jnp.concatenate(
        [hp, jnp.zeros((hp.shape[0], F_IN - C_OUT), jnp.float32)], axis=1)


def _tc_layer2_body(agg2p_ref, cnt_ref, h_ref,
                    b2_ref, w2r_ref, wlin_ref, blin_ref,
                    out_ref):
    cnt = cnt_ref[0, :, 0:1] + cnt_ref[1, :, 0:1]
    denom = jnp.maximum(cnt, 1.0)
    aggp = (agg2p_ref[0, :, :C_OUT] + agg2p_ref[1, :, :C_OUT]) / denom
    hcat = jnp.concatenate([h_ref[0], h_ref[1]], axis=1)
    wh = wlin_ref[...] + lax.dot_general(wlin_ref[...], w2r_ref[...],
                                         (((1,), (0,)), ((), ())),
                                         preferred_element_type=jnp.float32)
    bcomb = blin_ref[...] + _dotT(b2_ref[...], wlin_ref[...])
    logits = _dotT(hcat, wh) + aggp + bcomb
    m = jnp.max(logits, axis=1, keepdims=True)
    lse = jnp.log(jnp.sum(jnp.exp(logits - m), axis=1, keepdims=True)) + m
    out_ref[...] = logits - lse


def _whole(shape):
    return pl.BlockSpec(shape, lambda i: tuple(0 for _ in shape))


def kernel(x, edge_index, W1l, b1, W1r, W2l, b2, W2r, Wlin, blin):
    src = edge_index[0].astype(jnp.int32)
    dst = edge_index[1].astype(jnp.int32)

    agg1, cnt1 = _sc_agg_x(src, dst, x)

    b1r = b1.reshape(1, H)
    b2r = b2.reshape(1, H)
    blinr = blin.reshape(1, C_OUT)

    grid = (N // BR,)
    h2, hp = pl.pallas_call(
        _tc_layer1_body,
        grid=grid,
        in_specs=[
            pl.BlockSpec((NC, BR, F_IN), lambda i: (0, i, 0)),
            pl.BlockSpec((NC, BR, F_IN), lambda i: (0, i, 0)),
            pl.BlockSpec((BR, F_IN), lambda i: (i, 0)),
            _whole((H, F_IN)), _whole((1, H)), _whole((H, F_IN)),
            _whole((H, H)), _whole((C_OUT, H)),
        ],
        out_specs=[
            pl.BlockSpec((NC, BR, H // 2), lambda i: (0, i, 0)),
            pl.BlockSpec((BR, F_IN), lambda i: (i, 0)),
        ],
        out_shape=[
            jax.ShapeDtypeStruct((NC, N, H // 2), jnp.float32),
            jax.ShapeDtypeStruct((N, F_IN), jnp.float32),
        ],
    )(agg1, cnt1, x, W1l, b1r, W1r, W2l, Wlin)

    agg2p = _sc_agg_p(src, dst, hp)

    out = pl.pallas_call(
        _tc_layer2_body,
        grid=grid,
        in_specs=[
            pl.BlockSpec((NC, BR, F_IN), lambda i: (0, i, 0)),
            pl.BlockSpec((NC, BR, F_IN), lambda i: (0, i, 0)),
            pl.BlockSpec((NC, BR, H // 2), lambda i: (0, i, 0)),
            _whole((1, H)), _whole((H, H)),
            _whole((C_OUT, H)), _whole((1, C_OUT)),
        ],
        out_specs=pl.BlockSpec((BR, C_OUT), lambda i: (i, 0)),
        out_shape=jax.ShapeDtypeStruct((N, C_OUT), jnp.float32),
    )(agg2p, cnt1, h2, b2r, W2r, Wlin, blinr)

    return (out, edge_index)
